# initial kernel scaffold (unmeasured)
import jax
import jax.numpy as jnp
from jax import lax
from jax.experimental import pallas as pl
from jax.experimental.pallas import tpu as pltpu


def kernel(
    x,
):
    def body(*refs):
        pass

    out_shape = jax.ShapeDtypeStruct(..., jnp.float32)
    return pl.pallas_call(body, out_shape=out_shape)(...)


# baseline (device time: 36784 ns/iter reference)
import jax
import jax.numpy as jnp
from jax import lax
from jax.experimental import pallas as pl
from jax.experimental.pallas import tpu as pltpu


def kernel(x):
    _, m, n = x.shape
    half = n // 2

    def body(x_ref, out_ref, comm_ref, send_sem, recv_sem):
        my_x = lax.axis_index("x")
        my_y = lax.axis_index("y")
        my_z = lax.axis_index("z")
        peer_y = 1 - my_y

        comm_ref[0] = x_ref[0, :, pl.ds(peer_y * half, half)].astype(
            jnp.bfloat16
        )

        rdma = pltpu.make_async_remote_copy(
            src_ref=comm_ref.at[0],
            dst_ref=comm_ref.at[1],
            send_sem=send_sem,
            recv_sem=recv_sem,
            device_id=(my_x, peer_y, my_z),
            device_id_type=pl.DeviceIdType.MESH,
        )
        rdma.start()
        rdma.wait()

        mine = x_ref[0, :, pl.ds(my_y * half, half)]
        out_ref[...] = mine + comm_ref[1].astype(jnp.float32)

    return pl.pallas_call(
        body,
        out_shape=jax.ShapeDtypeStruct((m, half), jnp.float32),
        in_specs=[pl.BlockSpec(memory_space=pltpu.VMEM)],
        out_specs=pl.BlockSpec(memory_space=pltpu.VMEM),
        scratch_shapes=[
            pltpu.VMEM((2, m, half), jnp.bfloat16),
            pltpu.SemaphoreType.DMA,
            pltpu.SemaphoreType.DMA,
        ],
    )(x)


# device time: 32124 ns/iter; 1.1451x vs baseline; 1.1451x over previous
import jax
import jax.numpy as jnp
from jax import lax
from jax.experimental import pallas as pl
from jax.experimental.pallas import tpu as pltpu

NCHUNK = 4


def kernel(x):
    _, m, n = x.shape
    half = n // 2
    cw = half // NCHUNK

    def body(x_ref, out_ref, comm_ref, send_sems, recv_sems):
        my_x = lax.axis_index("x")
        my_y = lax.axis_index("y")
        my_z = lax.axis_index("z")
        peer_y = 1 - my_y
        peer = (my_x, peer_y, my_z)

        barrier_sem = pltpu.get_barrier_semaphore()
        pl.semaphore_signal(
            barrier_sem, inc=1, device_id=peer,
            device_id_type=pl.DeviceIdType.MESH,
        )
        pl.semaphore_wait(barrier_sem, 1)

        rdmas = []
        for c in range(NCHUNK):
            comm_ref[0, c] = x_ref[
                0, :, pl.ds(peer_y * half + c * cw, cw)
            ].astype(jnp.bfloat16)
            rdma = pltpu.make_async_remote_copy(
                src_ref=comm_ref.at[0, c],
                dst_ref=comm_ref.at[1, c],
                send_sem=send_sems.at[c],
                recv_sem=recv_sems.at[c],
                device_id=peer,
                device_id_type=pl.DeviceIdType.MESH,
            )
            rdma.start()
            rdmas.append(rdma)

        for c in range(NCHUNK):
            rdmas[c].wait_recv()
            mine = x_ref[0, :, pl.ds(my_y * half + c * cw, cw)]
            out_ref[:, c * cw:(c + 1) * cw] = (
                mine + comm_ref[1, c].astype(jnp.float32)
            )

        for c in range(NCHUNK):
            rdmas[c].wait_send()

    return pl.pallas_call(
        body,
        out_shape=jax.ShapeDtypeStruct((m, half), jnp.float32),
        in_specs=[pl.BlockSpec(memory_space=pltpu.VMEM)],
        out_specs=pl.BlockSpec(memory_space=pltpu.VMEM),
        scratch_shapes=[
            pltpu.VMEM((2, NCHUNK, m, cw), jnp.bfloat16),
            pltpu.SemaphoreType.DMA((NCHUNK,)),
            pltpu.SemaphoreType.DMA((NCHUNK,)),
        ],
        compiler_params=pltpu.CompilerParams(collective_id=0),
    )(x)


# device time: 28375 ns/iter; 1.2964x vs baseline; 1.1321x over previous
import jax
import jax.numpy as jnp
from jax import lax
from jax.experimental import pallas as pl
from jax.experimental.pallas import tpu as pltpu

MESH = pl.DeviceIdType.MESH


def kernel(x):
    _, m, n = x.shape
    half = n // 2
    qw = half // 4
    mh = m // 2

    def body(x_ref, out_ref, s1s, s1r, s2x, s2z, s3, send_sems, recv_sems):
        mx = lax.axis_index("x")
        my = lax.axis_index("y")
        mz = lax.axis_index("z")
        peer_y = (mx, 1 - my, mz)
        nbr_x = (1 - mx, my, mz)
        nbr_z = (mx, my, 1 - mz)

        my_base = my * half
        peer_base = (1 - my) * half
        q_own = 2 * mx + mz
        q_x = 2 * (1 - mx) + mz
        q_z = 2 * mx + (1 - mz)
        q_d = 2 * (1 - mx) + (1 - mz)

        barrier_sem = pltpu.get_barrier_semaphore()
        for nbr in (peer_y, nbr_x, nbr_z):
            pl.semaphore_signal(barrier_sem, inc=1, device_id=nbr,
                                device_id_type=MESH)
        pl.semaphore_wait(barrier_sem, 3)

        def exchange(src, dst, sem_idx, dev):
            r = pltpu.make_async_remote_copy(
                src_ref=src, dst_ref=dst,
                send_sem=send_sems.at[sem_idx],
                recv_sem=recv_sems.at[sem_idx],
                device_id=dev, device_id_type=MESH,
            )
            r.start()
            return r

        def add_quarter(q, contrib):
            mine = x_ref[0, :, pl.ds(my_base + q * qw, qw)]
            out_ref[:, pl.ds(q * qw, qw)] = mine + contrib.astype(jnp.float32)

        s1s[...] = x_ref[0, :, pl.ds(peer_base + q_own * qw, qw)].astype(
            jnp.bfloat16
        ).reshape(2, mh, qw)
        r1 = exchange(s1s, s1r, 0, peer_y)
        r1.wait_recv()

        r2x = exchange(s1r, s2x, 1, nbr_x)
        r2z = exchange(s1r, s2z, 2, nbr_z)

        add_quarter(q_own, s1r[...].reshape(m, qw))

        r2z.wait_recv()
        r3x = exchange(s2z.at[0], s3.at[0], 3, nbr_x)
        r2x.wait_recv()
        r3z = exchange(s2x.at[1], s3.at[1], 4, nbr_z)

        add_quarter(q_x, s2x[...].reshape(m, qw))
        add_quarter(q_z, s2z[...].reshape(m, qw))

        r3x.wait_recv()
        r3z.wait_recv()
        add_quarter(q_d, s3[...].reshape(m, qw))

        for r in (r1, r2x, r2z, r3x, r3z):
            r.wait_send()

    buf = pltpu.VMEM((2, mh, qw), jnp.bfloat16)
    return pl.pallas_call(
        body,
        out_shape=jax.ShapeDtypeStruct((m, half), jnp.float32),
        in_specs=[pl.BlockSpec(memory_space=pltpu.VMEM)],
        out_specs=pl.BlockSpec(memory_space=pltpu.VMEM),
        scratch_shapes=[
            buf,
            buf,
            buf,
            buf,
            buf,
            pltpu.SemaphoreType.DMA((5,)),
            pltpu.SemaphoreType.DMA((5,)),
        ],
        compiler_params=pltpu.CompilerParams(collective_id=0),
    )(x)


# device time: 27756 ns/iter; 1.3253x vs baseline; 1.0223x over previous
import jax
import jax.numpy as jnp
from jax import lax
from jax.experimental import pallas as pl
from jax.experimental.pallas import tpu as pltpu

MESH = pl.DeviceIdType.MESH


def kernel(x):
    _, m, n = x.shape
    half = n // 2
    qw = half // 4
    mh = m // 2

    def body(x_ref, out_ref, s1s, s1r, s2x, s2z, s3, send_sems, recv_sems):
        mx = lax.axis_index("x")
        my = lax.axis_index("y")
        mz = lax.axis_index("z")
        peer_y = (mx, 1 - my, mz)
        nbr_x = (1 - mx, my, mz)
        nbr_z = (mx, my, 1 - mz)

        my_base = my * half
        peer_base = (1 - my) * half
        q_own = 2 * mx + mz
        q_x = 2 * (1 - mx) + mz
        q_z = 2 * mx + (1 - mz)
        q_d = 2 * (1 - mx) + (1 - mz)

        barrier_sem = pltpu.get_barrier_semaphore()
        for nbr in (peer_y, nbr_x, nbr_z):
            pl.semaphore_signal(barrier_sem, inc=1, device_id=nbr,
                                device_id_type=MESH)
        pl.semaphore_wait(barrier_sem, 3)

        def exchange(src, dst, sem_idx, dev):
            r = pltpu.make_async_remote_copy(
                src_ref=src, dst_ref=dst,
                send_sem=send_sems.at[sem_idx],
                recv_sem=recv_sems.at[sem_idx],
                device_id=dev, device_id_type=MESH,
            )
            r.start()
            return r

        def add_quarter(q, contrib):
            mine = x_ref[0, :, pl.ds(my_base + q * qw, qw)]
            out_ref[:, pl.ds(q * qw, qw)] = (
                mine + contrib.astype(jnp.float32)
            ).astype(jnp.bfloat16)

        s1s[...] = x_ref[0, :, pl.ds(peer_base + q_own * qw, qw)].astype(
            jnp.bfloat16
        ).reshape(2, mh, qw)
        r1 = exchange(s1s, s1r, 0, peer_y)
        r1.wait_recv()

        r2x = exchange(s1r, s2x, 1, nbr_x)
        r2z = exchange(s1r, s2z, 2, nbr_z)

        add_quarter(q_own, s1r[...].reshape(m, qw))

        r2z.wait_recv()
        r3x = exchange(s2z.at[0], s3.at[0], 3, nbr_x)
        r2x.wait_recv()
        r3z = exchange(s2x.at[1], s3.at[1], 4, nbr_z)

        add_quarter(q_x, s2x[...].reshape(m, qw))
        add_quarter(q_z, s2z[...].reshape(m, qw))

        r3x.wait_recv()
        r3z.wait_recv()
        add_quarter(q_d, s3[...].reshape(m, qw))

        for r in (r1, r2x, r2z, r3x, r3z):
            r.wait_send()

    buf = pltpu.VMEM((2, mh, qw), jnp.bfloat16)
    return pl.pallas_call(
        body,
        out_shape=jax.ShapeDtypeStruct((m, half), jnp.bfloat16),
        in_specs=[pl.BlockSpec(memory_space=pltpu.VMEM)],
        out_specs=pl.BlockSpec(memory_space=pltpu.VMEM),
        scratch_shapes=[
            buf,
            buf,
            buf,
            buf,
            buf,
            pltpu.SemaphoreType.DMA((5,)),
            pltpu.SemaphoreType.DMA((5,)),
        ],
        compiler_params=pltpu.CompilerParams(collective_id=0),
    )(x)


# device time: 21814 ns/iter; 1.6863x vs baseline; 1.2724x over previous
import jax
import jax.numpy as jnp
from jax import lax
from jax.experimental import pallas as pl
from jax.experimental.pallas import tpu as pltpu

MESH = pl.DeviceIdType.MESH


def kernel(x):
    _, m, n = x.shape
    half = n // 2
    qw = half // 4
    mh = m // 2

    def body(x_ref, out_ref, ysend, yrecv, x2recv, z2recv,
             send_sems, recv_sems):
        mx = lax.axis_index("x")
        my = lax.axis_index("y")
        mz = lax.axis_index("z")
        peer_y = (mx, 1 - my, mz)
        nbr_x = (1 - mx, my, mz)
        nbr_z = (mx, my, 1 - mz)

        my_base = my * half
        peer_base = (1 - my) * half
        q_own = 2 * mx + mz
        q_x = 2 * (1 - mx) + mz
        q_z = 2 * mx + (1 - mz)
        q_d = 2 * (1 - mx) + (1 - mz)

        barrier_sem = pltpu.get_barrier_semaphore()
        for nbr in (peer_y, nbr_x, nbr_z):
            pl.semaphore_signal(barrier_sem, inc=1, device_id=nbr,
                                device_id_type=MESH)
        pl.semaphore_wait(barrier_sem, 3)

        def exchange(src, dst, sem_idx, dev):
            r = pltpu.make_async_remote_copy(
                src_ref=src, dst_ref=dst,
                send_sem=send_sems.at[sem_idx],
                recv_sem=recv_sems.at[sem_idx],
                device_id=dev, device_id_type=MESH,
            )
            r.start()
            return r

        def add_quarter_half(q, h, contrib):
            rows = pl.ds(h * mh, mh)
            mine = x_ref[0, rows, pl.ds(my_base + q * qw, qw)]
            out_ref[rows, pl.ds(q * qw, qw)] = (
                mine + contrib.astype(jnp.float32)
            ).astype(jnp.bfloat16)

        ry = []
        for qi, col_q in ((0, q_own), (1, q_d)):
            for h in (0, 1):
                ysend[qi, h] = x_ref[
                    0, pl.ds(h * mh, mh), pl.ds(peer_base + col_q * qw, qw)
                ].astype(jnp.bfloat16)
                ry.append(
                    exchange(ysend.at[qi, h], yrecv.at[qi, h],
                             2 * qi + h, peer_y)
                )

        rx, rz = [], []
        for h in (0, 1):
            ry[h].wait_recv()
            rx.append(exchange(yrecv.at[0, h], x2recv.at[h], 4 + h, nbr_x))
            rz.append(exchange(yrecv.at[0, h], z2recv.at[h], 6 + h, nbr_z))
            add_quarter_half(q_own, h, yrecv[0, h])

        for h in (0, 1):
            ry[2 + h].wait_recv()
            add_quarter_half(q_d, h, yrecv[1, h])

        for h in (0, 1):
            rx[h].wait_recv()
            add_quarter_half(q_x, h, x2recv[h])
        for h in (0, 1):
            rz[h].wait_recv()
            add_quarter_half(q_z, h, z2recv[h])

        for r in ry + rx + rz:
            r.wait_send()

    return pl.pallas_call(
        body,
        out_shape=jax.ShapeDtypeStruct((m, half), jnp.bfloat16),
        in_specs=[pl.BlockSpec(memory_space=pltpu.VMEM)],
        out_specs=pl.BlockSpec(memory_space=pltpu.VMEM),
        scratch_shapes=[
            pltpu.VMEM((2, 2, mh, qw), jnp.bfloat16),
            pltpu.VMEM((2, 2, mh, qw), jnp.bfloat16),
            pltpu.VMEM((2, mh, qw), jnp.bfloat16),
            pltpu.VMEM((2, mh, qw), jnp.bfloat16),
            pltpu.SemaphoreType.DMA((8,)),
            pltpu.SemaphoreType.DMA((8,)),
        ],
        compiler_params=pltpu.CompilerParams(collective_id=0),
    )(x)
